# baseline (speedup 1.0000x reference)
import functools

import jax
import jax.numpy as jnp
from jax.experimental import pallas as pl
from jax.experimental.pallas import tpu as pltpu

_B, _S, _D = 4, 2048, 1024
_T = _S // 2
_ROWS = 1024
_NT = _S // _ROWS


def _flow_kernel(sent_ref, ent_ref, w_ref, bias_ref, sent_any, ent_any,
                 hat_out, a_out, loss_out,
                 loss_acc, cnt, best_row, srows, erow, sem0, sem1):
    b = pl.program_id(0)
    t = pl.program_id(1)

    @pl.when(jnp.logical_and(b == 0, t == 0))
    def _init_loss():
        loss_out[...] = jnp.zeros((1, 128), jnp.float32)

    @pl.when(t == 0)
    def _init_batch():
        loss_acc[0] = 0.0
        cnt[0] = 0
        best_row[0] = _S - 2

    x = sent_ref[0]
    e = ent_ref[0]

    mm = jax.lax.dot_general(
        e.astype(jnp.bfloat16), w_ref[...],
        dimension_numbers=(((1,), (0,)), ((), ())),
        preferred_element_type=jnp.float32)
    g_s = pltpu.roll(mm - x, _ROWS - 1, 0)
    diff = x + g_s + bias_ref[...]

    rowsum = jnp.sum(x, axis=1, keepdims=True)
    ids = jax.lax.broadcasted_iota(jnp.int32, (_ROWS, 1), 0)
    maskv = (rowsum != 0.0) & (ids % 2 == 0)
    d2row = jnp.sum(diff * diff, axis=1, keepdims=True)
    loss_acc[0] += jnp.sum(jnp.where(maskv, d2row, 0.0))

    tile_cnt = jnp.sum(maskv.astype(jnp.int32))
    cnt[0] += tile_cnt

    @pl.when(tile_cnt > 0)
    def _track_last():
        tl = jnp.max(jnp.where(maskv, ids, -1))
        best_row[0] = t * _ROWS + tl

    @pl.when(t == _NT - 1)
    def _finish_batch():
        g = best_row[0]
        g8 = (g // 8) * 8
        off = g - g8
        cp_s = pltpu.make_async_copy(
            sent_any.at[b, pl.ds(g8, 8), :], srows, sem0)
        cp_e = pltpu.make_async_copy(
            ent_any.at[b, pl.ds(g8, 8), :], erow, sem1)
        cp_s.start()
        cp_e.start()
        cp_s.wait()
        cp_e.wait()
        q_row = srows[pl.ds(off, 1), :]
        a_row = srows[pl.ds(off + 1, 1), :]
        mm_row = jax.lax.dot_general(
            erow[pl.ds(off + 1, 1), :].astype(jnp.bfloat16), w_ref[...],
            dimension_numbers=(((1,), (0,)), ((), ())),
            preferred_element_type=jnp.float32)
        hat_row = q_row + mm_row + bias_ref[...]
        hat_out[0, 0, :] = hat_row[0]
        a_out[0, 0, :] = a_row[0]
        dd = hat_row - a_row
        d2 = jnp.where(cnt[0] > 0, jnp.sum(dd * dd), 0.0)
        loss_out[...] = loss_out[...] + (loss_acc[0] - d2)


@functools.partial(jax.jit, static_argnames=())
def kernel(sent_emb, entity_emb, W, b):
    bias = b.reshape(1, _D)
    wt_bf16 = W.T.astype(jnp.bfloat16)

    hat_n, a_n, loss = pl.pallas_call(
        _flow_kernel,
        grid=(_B, _NT),
        in_specs=[
            pl.BlockSpec((1, _ROWS, _D), lambda b_, t_: (b_, t_, 0)),
            pl.BlockSpec((1, _ROWS, _D), lambda b_, t_: (b_, t_, 0)),
            pl.BlockSpec((_D, _D), lambda b_, t_: (0, 0)),
            pl.BlockSpec((1, _D), lambda b_, t_: (0, 0)),
            pl.BlockSpec(memory_space=pl.ANY),
            pl.BlockSpec(memory_space=pl.ANY),
        ],
        out_specs=[
            pl.BlockSpec((1, 1, _D), lambda b_, t_: (b_, 0, 0)),
            pl.BlockSpec((1, 1, _D), lambda b_, t_: (b_, 0, 0)),
            pl.BlockSpec((1, 128), lambda b_, t_: (0, 0)),
        ],
        out_shape=[
            jax.ShapeDtypeStruct((_B, 1, _D), jnp.float32),
            jax.ShapeDtypeStruct((_B, 1, _D), jnp.float32),
            jax.ShapeDtypeStruct((1, 128), jnp.float32),
        ],
        scratch_shapes=[
            pltpu.SMEM((1,), jnp.float32),
            pltpu.SMEM((1,), jnp.int32),
            pltpu.SMEM((1,), jnp.int32),
            pltpu.VMEM((8, _D), jnp.float32),
            pltpu.VMEM((8, _D), jnp.float32),
            pltpu.SemaphoreType.DMA,
            pltpu.SemaphoreType.DMA,
        ],
    )(sent_emb, entity_emb, wt_bf16, bias, sent_emb, entity_emb)

    return (hat_n[:, 0, :], a_n[:, 0, :], loss[0, 0])
